# Initial kernel scaffold; baseline (speedup 1.0000x reference)
import jax
import jax.numpy as jnp
from jax.experimental import pallas as pl


def kernel(x, mu, log_sigma, W0, W1, W2, W3, W4, W5, W6, W7, W8):
    raise NotImplementedError("write your pallas kernel here")


# speedup vs baseline: 5.9348x; 5.9348x over previous
import functools
import math

import jax
import jax.numpy as jnp
from jax.experimental import pallas as pl
from jax.experimental.pallas import tpu as pltpu

D = 512
K = 16
LEVELS = 9
_LOG2PI = math.log(2.0 * math.pi)


def _circuit_kernel(xt_ref, mu_ref, ls_ref, *refs):
    w_refs = refs[:LEVELS]
    o_ref = refs[LEVELS]
    bt = xt_ref.shape[1]

    x = xt_ref[...]
    mu = mu_ref[...]
    ls = ls_ref[...]

    inv_sigma = jnp.exp(-ls)
    z = (x[:, None, :] - mu[:, :, None]) * inv_sigma[:, :, None]
    state4 = -0.5 * z * z - ls[:, :, None] - 0.5 * _LOG2PI
    state = state4.reshape(D * K, bt)

    f = D
    for l in range(LEVELS):
        f2 = f // 2
        w = w_refs[l][...]
        s4 = state.reshape(f2, 2, K, bt)
        prod = s4[:, 0, :, :] + s4[:, 1, :, :]
        m = jnp.max(prod, axis=1, keepdims=True)
        e = jnp.exp(prod - m).reshape(f2 * K, bt)
        wmax = jnp.max(w, axis=-1, keepdims=True)
        ew = jnp.exp(w - wmax)
        expw = ew / jnp.sum(ew, axis=-1, keepdims=True)
        g = min(f2, 8)
        w2 = expw.reshape(f2 * K, K)
        mt = jnp.concatenate([w2] * g, axis=1)
        r = jax.lax.broadcasted_iota(jnp.int32, (f2 * K, g * K), 0)
        c = jax.lax.broadcasted_iota(jnp.int32, (f2 * K, g * K), 1)
        bd = jnp.where((c // K) == ((r // K) % g), mt, 0.0)
        ng = f2 // g
        rows = g * K
        outs = []
        for gi in range(ng):
            bd_g = bd[gi * rows:(gi + 1) * rows, :]
            e_g = e[gi * rows:(gi + 1) * rows, :]
            outs.append(jax.lax.dot_general(
                bd_g, e_g, (((1,), (0,)), ((), ())),
                preferred_element_type=jnp.float32))
        y = outs[0] if ng == 1 else jnp.concatenate(outs, axis=0)
        state = (jnp.log(y).reshape(f2, K, bt) + m).reshape(f2 * K, bt)
        f = f2

    o_ref[...] = state


@functools.partial(jax.jit, static_argnames=("bt",))
def _run(xt, mu, log_sigma, ws, bt):
    b = xt.shape[1]
    nt = b // bt
    w_specs = [pl.BlockSpec(w.shape, lambda i: (0, 0, 0)) for w in ws]
    out = pl.pallas_call(
        _circuit_kernel,
        grid=(nt,),
        in_specs=[
            pl.BlockSpec((D, bt), lambda i: (0, i)),
            pl.BlockSpec((D, K), lambda i: (0, 0)),
            pl.BlockSpec((D, K), lambda i: (0, 0)),
            *w_specs,
        ],
        out_specs=pl.BlockSpec((K, bt), lambda i: (0, i)),
        out_shape=jax.ShapeDtypeStruct((K, b), jnp.float32),
        compiler_params=pltpu.CompilerParams(
            dimension_semantics=("arbitrary",)),
    )(xt, mu, log_sigma, *ws)
    return out


def kernel(x, mu, log_sigma, W0, W1, W2, W3, W4, W5, W6, W7, W8):
    b = x.shape[0]
    xt = x[:, 0, :].T
    ws = [W0, W1, W2, W3, W4, W5, W6, W7, W8]
    out = _run(xt, mu, log_sigma, ws, bt=256)
    return out.T.reshape(b, 1, K)


# speedup vs baseline: 9.1315x; 1.5386x over previous
import functools
import math

import jax
import jax.numpy as jnp
from jax.experimental import pallas as pl
from jax.experimental.pallas import tpu as pltpu

D = 512
K = 16
LEVELS = 9
_LOG2PI = math.log(2.0 * math.pi)
_FOLDS = [D // 2 ** (l + 1) for l in range(LEVELS)]
_GROUPS = [min(f2, 8) for f2 in _FOLDS]


def _prep_kernel(mu_ref, ls_ref, *refs):
    w_refs = refs[:LEVELS]
    ga_ref, gb_ref, gc_ref = refs[LEVELS:LEVELS + 3]
    bd_refs = refs[LEVELS + 3:]

    mu = mu_ref[...]
    ls = ls_ref[...]
    isig2 = jnp.exp(-2.0 * ls)
    ga_ref[...] = -0.5 * isig2
    gb_ref[...] = mu * isig2
    gc_ref[...] = -0.5 * mu * mu * isig2 - ls - 0.5 * _LOG2PI

    for l in range(LEVELS):
        f2 = _FOLDS[l]
        g = _GROUPS[l]
        w = w_refs[l][...]
        wmax = jnp.max(w, axis=-1, keepdims=True)
        ew = jnp.exp(w - wmax)
        expw = ew / jnp.sum(ew, axis=-1, keepdims=True)
        w2 = expw.reshape(f2 * K, K)
        mt = jnp.concatenate([w2] * g, axis=1)
        r = jax.lax.broadcasted_iota(jnp.int32, (f2 * K, g * K), 0)
        c = jax.lax.broadcasted_iota(jnp.int32, (f2 * K, g * K), 1)
        bd_refs[l][...] = jnp.where((c // K) == ((r // K) % g), mt, 0.0)


def _circuit_kernel(xt_ref, ga_ref, gb_ref, gc_ref, *refs):
    bd_refs = refs[:LEVELS]
    o_ref = refs[LEVELS]
    bt = xt_ref.shape[1]

    x = xt_ref[...]
    x2 = (x * x)[:, None, :]
    xb = x[:, None, :]
    t = (ga_ref[...][:, :, None] * x2
         + gb_ref[...][:, :, None] * xb
         + gc_ref[...][:, :, None])
    state = t.reshape(D * K, bt)

    for l in range(LEVELS):
        f2 = _FOLDS[l]
        g = _GROUPS[l]
        s4 = state.reshape(f2, 2, K, bt)
        prod = s4[:, 0, :, :] + s4[:, 1, :, :]
        m = jnp.max(prod, axis=1, keepdims=True)
        e = jnp.exp(prod - m).reshape(f2 * K, bt)
        bd = bd_refs[l][...]
        ng = f2 // g
        rows = g * K
        outs = []
        for gi in range(ng):
            outs.append(jax.lax.dot_general(
                bd[gi * rows:(gi + 1) * rows, :],
                e[gi * rows:(gi + 1) * rows, :],
                (((1,), (0,)), ((), ())),
                preferred_element_type=jnp.float32))
        y = outs[0] if ng == 1 else jnp.concatenate(outs, axis=0)
        state = (jnp.log(y).reshape(f2, K, bt) + m).reshape(f2 * K, bt)

    o_ref[...] = state


@functools.partial(jax.jit, static_argnames=("bt",))
def _run(xt, mu, log_sigma, ws, bt):
    b = xt.shape[1]
    nt = b // bt

    prep_out = pl.pallas_call(
        _prep_kernel,
        out_shape=(
            jax.ShapeDtypeStruct((D, K), jnp.float32),
            jax.ShapeDtypeStruct((D, K), jnp.float32),
            jax.ShapeDtypeStruct((D, K), jnp.float32),
            *[jax.ShapeDtypeStruct((f2 * K, g * K), jnp.float32)
              for f2, g in zip(_FOLDS, _GROUPS)],
        ),
    )(mu, log_sigma, *ws)
    ga, gb, gc = prep_out[:3]
    bds = prep_out[3:]

    bd_specs = [pl.BlockSpec(a.shape, lambda i: (0, 0)) for a in bds]
    out = pl.pallas_call(
        _circuit_kernel,
        grid=(nt,),
        in_specs=[
            pl.BlockSpec((D, bt), lambda i: (0, i)),
            pl.BlockSpec((D, K), lambda i: (0, 0)),
            pl.BlockSpec((D, K), lambda i: (0, 0)),
            pl.BlockSpec((D, K), lambda i: (0, 0)),
            *bd_specs,
        ],
        out_specs=pl.BlockSpec((K, bt), lambda i: (0, i)),
        out_shape=jax.ShapeDtypeStruct((K, b), jnp.float32),
        compiler_params=pltpu.CompilerParams(
            dimension_semantics=("arbitrary",)),
    )(xt, ga, gb, gc, *bds)
    return out


def kernel(x, mu, log_sigma, W0, W1, W2, W3, W4, W5, W6, W7, W8):
    b = x.shape[0]
    xt = x[:, 0, :].T
    ws = [W0, W1, W2, W3, W4, W5, W6, W7, W8]
    out = _run(xt, mu, log_sigma, ws, bt=256)
    return out.T.reshape(b, 1, K)
